# Initial kernel scaffold; baseline (speedup 1.0000x reference)
import jax
import jax.numpy as jnp
from jax.experimental import pallas as pl


def kernel(input, weight_indices, weight_values, bias_indices, bias_values):
    raise NotImplementedError("write your pallas kernel here")


# speedup vs baseline: 2.8112x; 2.8112x over previous
import functools

import jax
import jax.numpy as jnp
from jax import lax
from jax.experimental import pallas as pl
from jax.experimental.pallas import tpu as pltpu
from jax.experimental.pallas import tpu_sc as plsc

OUT_DIM = 4096
NC = 2
NS = 16
LANES = 16
CHUNK = 16


def _spmm_kernel(in_dim1, bh, n_per_tile,
                 stacked_hbm, rows_hbm, cols_hbm, vals_hbm, out_hbm,
                 rows_v, cols_v, vals_v, gbuf, sbuf, acc, gsem):
    cid = lax.axis_index("c")
    sid = lax.axis_index("s")

    base = sid * n_per_tile
    pltpu.sync_copy(rows_hbm.at[pl.ds(base, n_per_tile)], rows_v)
    pltpu.sync_copy(cols_hbm.at[pl.ds(base, n_per_tile)], cols_v)
    pltpu.sync_copy(vals_hbm.at[pl.ds(base, n_per_tile)], vals_v)

    zero16 = jnp.zeros((LANES,), jnp.float32)
    for j in range(CHUNK):
        for q in range(bh // LANES):
            sbuf[j, pl.ds(q * LANES, LANES)] = zero16
    rows_per_tile = OUT_DIM // NS
    for t in range(rows_per_tile // CHUNK):
        pltpu.sync_copy(sbuf, acc.at[pl.ds(sid * rows_per_tile + t * CHUNK, CHUNK)])
    plsc.subcore_barrier()

    col_off = cid * in_dim1

    def body(i, carry):
        off = i * CHUNK
        cvec = cols_v[pl.ds(off, CHUNK)] + col_off
        rvec = rows_v[pl.ds(off, CHUNK)]
        vvec = vals_v[pl.ds(off, CHUNK)]
        pltpu.async_copy(stacked_hbm.at[cvec], gbuf, gsem).wait()
        for j in range(CHUNK):
            s = jnp.full((LANES,), vvec[j], jnp.float32)
            for q in range(bh // LANES):
                sl = pl.ds(q * LANES, LANES)
                sbuf[j, sl] = gbuf[j, sl] * s
        pltpu.sync_copy(sbuf, acc.at[rvec], add=True)
        return carry

    lax.fori_loop(0, n_per_tile // CHUNK, body, 0)

    plsc.subcore_barrier()
    pltpu.sync_copy(
        acc.at[pl.ds(sid * rows_per_tile, rows_per_tile)],
        out_hbm.at[pl.ds(cid * OUT_DIM + sid * rows_per_tile, rows_per_tile)])


def kernel(input, weight_indices, weight_values, bias_indices, bias_values):
    b, in_dim = input.shape
    bh = b // NC
    in_dim1 = in_dim + 1

    inp2 = jnp.concatenate(
        [input, jnp.ones((b, 1), input.dtype)], axis=1).T
    stacked = jnp.concatenate([inp2[:, :bh], inp2[:, bh:]], axis=0)

    rows_all = jnp.concatenate([weight_indices[0], bias_indices[0]])
    cols_all = jnp.concatenate(
        [weight_indices[1], jnp.full(bias_values.shape, in_dim, jnp.int32)])
    vals_all = jnp.concatenate([weight_values, bias_values])

    n = rows_all.shape[0]
    n_per_tile = -(-n // NS)
    n_per_tile = -(-n_per_tile // CHUNK) * CHUNK
    n_pad = NS * n_per_tile
    pad = n_pad - n
    rows_all = jnp.pad(rows_all, (0, pad))
    cols_all = jnp.pad(cols_all, (0, pad))
    vals_all = jnp.pad(vals_all, (0, pad))

    mesh = plsc.VectorSubcoreMesh(
        core_axis_name="c", subcore_axis_name="s",
        num_cores=NC, num_subcores=NS)
    out_t = pl.kernel(
        functools.partial(_spmm_kernel, in_dim1, bh, n_per_tile),
        out_type=jax.ShapeDtypeStruct((NC * OUT_DIM, bh), jnp.float32),
        mesh=mesh,
        scratch_types=[
            pltpu.VMEM((n_per_tile,), jnp.int32),
            pltpu.VMEM((n_per_tile,), jnp.int32),
            pltpu.VMEM((n_per_tile,), jnp.float32),
            pltpu.VMEM((CHUNK, bh), jnp.float32),
            pltpu.VMEM((CHUNK, bh), jnp.float32),
            pltpu.VMEM_SHARED((OUT_DIM, bh), jnp.float32),
            pltpu.SemaphoreType.DMA,
        ],
    )(stacked, rows_all, cols_all, vals_all)

    return jnp.concatenate([out_t[:OUT_DIM].T, out_t[OUT_DIM:].T], axis=0)
